# baseline (speedup 1.0000x reference)
import functools

import jax
import jax.numpy as jnp
from jax import lax
from jax.experimental import pallas as pl
from jax.experimental.pallas import tpu as pltpu
from jax.experimental.pallas import tpu_sc as plsc

_NC = 2
_NS = 16
_NW = _NC * _NS
_D = 128
_DH = 64
_CW = 32

_N1, _N2 = 10000, 2000
_NPAD1, _NPAD2 = 10112, 2048


def _make_sc_agg(E, NPAD, B, K):
    e_per_t = E // _NS
    nb = e_per_t // B
    ns = nb // K
    assert nb * B == e_per_t and B % 16 == 0 and B <= 128 and ns * K == nb
    rpt = NPAD // _NS
    ZR = rpt // 2
    assert rpt % ZR == 0 and NPAD % 16 == 0 and ZR % 4 == 0
    mesh = plsc.VectorSubcoreMesh(core_axis_name="c", subcore_axis_name="s")

    @functools.partial(
        pl.kernel,
        out_type=jax.ShapeDtypeStruct((_NC * NPAD, _DH), jnp.float32),
        mesh=mesh,
        compiler_params=pltpu.CompilerParams(use_tc_tiling_on_sc=False),
        scratch_types=[
            pltpu.VMEM((K, B), jnp.int32),
            pltpu.VMEM((K, B), jnp.int32),
            pltpu.VMEM((K, B), jnp.int32),
            pltpu.VMEM((K, B, _DH), jnp.float32),
            pltpu.VMEM((ZR, _DH), jnp.float32),
            pltpu.VMEM_SHARED((NPAD, _DH), jnp.float32),
            pltpu.SemaphoreType.DMA,
            pltpu.SemaphoreType.DMA,
        ],
    )
    def agg(tab, src, dst, out_acc, idx_s, idx_d, idx_g, rows, zrow,
            acc_sh, sem_g, sem_s):
        c = lax.axis_index("c")
        s = lax.axis_index("s")

        z16 = jnp.zeros((16,), jnp.float32)

        def zr_body(i, carry):
            zrow[i // 4, pl.ds((i % 4) * 16, 16)] = z16
            return carry

        lax.fori_loop(0, ZR * 4, zr_body, 0)

        for k in range(rpt // ZR):
            pltpu.sync_copy(zrow, acc_sh.at[pl.ds(s * rpt + k * ZR, ZR)])
        plsc.subcore_barrier()

        row0 = s * nb

        def body(i, carry):
            r = row0 + i * K
            pltpu.sync_copy(src.at[pl.ds(r, K)], idx_s)
            pltpu.sync_copy(dst.at[pl.ds(r, K)], idx_d)
            for k in range(K):
                for j in range(B // 16):
                    v = idx_s[k, pl.ds(j * 16, 16)]
                    idx_g[k, pl.ds(j * 16, 16)] = v * 2 + c
            gets = [pltpu.async_copy(tab.at[idx_g.at[k]], rows.at[k], sem_g)
                    for k in range(K)]
            puts = []
            for k in range(K):
                gets[k].wait()
                puts.append(pltpu.async_copy(
                    rows.at[k], acc_sh.at[idx_d.at[k]], sem_s, add=True))
            for p in puts:
                p.wait()
            return carry

        lax.fori_loop(0, ns, body, 0)
        plsc.subcore_barrier()

        pltpu.sync_copy(acc_sh.at[pl.ds(s * rpt, rpt)],
                        out_acc.at[pl.ds(c * NPAD + s * rpt, rpt)])

    return agg


def _make_sc_cnt(E, NPAD, B, K):
    e_per_w = E // _NW
    nb = e_per_w // B
    ns = nb // K
    assert nb * B == e_per_w and B % 16 == 0 and B <= 128 and ns * K == nb
    rpt = NPAD // _NS
    mesh = plsc.VectorSubcoreMesh(core_axis_name="c", subcore_axis_name="s")

    @functools.partial(
        pl.kernel,
        out_type=jax.ShapeDtypeStruct((_NC * NPAD, _CW), jnp.float32),
        mesh=mesh,
        compiler_params=pltpu.CompilerParams(use_tc_tiling_on_sc=False),
        scratch_types=[
            pltpu.VMEM((K, B), jnp.int32),
            pltpu.VMEM((B, _CW), jnp.float32),
            pltpu.VMEM((rpt, _CW), jnp.float32),
            pltpu.VMEM_SHARED((NPAD, _CW), jnp.float32),
            pltpu.SemaphoreType.DMA,
        ],
    )
    def cnt(dst, out_cnt, idx_d, ones2, zc, cnt_sh, sem_s):
        c = lax.axis_index("c")
        s = lax.axis_index("s")
        wid = s * _NC + c

        z16 = jnp.zeros((16,), jnp.float32)
        one16 = jnp.ones((16,), jnp.float32)

        def zb(i, carry):
            zc[i, pl.ds(0, 16)] = z16
            zc[i, pl.ds(16, 16)] = z16
            return carry

        lax.fori_loop(0, rpt, zb, 0)

        def ob(i, carry):
            ones2[i, pl.ds(0, 16)] = one16
            ones2[i, pl.ds(16, 16)] = one16
            return carry

        lax.fori_loop(0, B, ob, 0)

        pltpu.sync_copy(zc, cnt_sh.at[pl.ds(s * rpt, rpt)])
        plsc.subcore_barrier()

        row0 = wid * nb

        def body(i, carry):
            pltpu.sync_copy(dst.at[pl.ds(row0 + i * K, K)], idx_d)
            puts = [pltpu.async_copy(ones2, cnt_sh.at[idx_d.at[k]], sem_s,
                                     add=True)
                    for k in range(K)]
            for p in puts:
                p.wait()
            return carry

        lax.fori_loop(0, ns, body, 0)
        plsc.subcore_barrier()

        pltpu.sync_copy(cnt_sh.at[pl.ds(s * rpt, rpt)],
                        out_cnt.at[pl.ds(c * NPAD + s * rpt, rpt)])

    return cnt


_E1_PAD = 327680
_E2_PAD = 163840
_B = 128
_sc_agg1 = _make_sc_agg(_E1_PAD, _NPAD1, _B, 8)
_sc_agg2 = _make_sc_agg(_E2_PAD, _NPAD2, _B, 8)
_sc_cnt1 = _make_sc_cnt(_E1_PAD, _NPAD1, _B, 8)
_sc_cnt2 = _make_sc_cnt(_E2_PAD, _NPAD2, _B, 8)


def _tc1_body(acc_ref, cnt_ref, w_ref, b_ref, out_ref):
    a = jnp.concatenate([acc_ref[0], acc_ref[1]], axis=1)
    cn = cnt_ref[0, :, 0:1] + cnt_ref[1, :, 0:1]
    m = a / jnp.maximum(cn, 1.0)
    y = lax.dot_general(m, w_ref[...], (((1,), (1,)), ((), ())),
                        preferred_element_type=jnp.float32)
    out_ref[...] = jnp.maximum(y + b_ref[...], 0.0)


def _tc2_body(acc_ref, cnt_ref, w_ref, b_ref, out_ref):
    a = jnp.concatenate([acc_ref[0], acc_ref[1]], axis=1)
    cn = cnt_ref[0, :, 0:1] + cnt_ref[1, :, 0:1]
    m = a / jnp.maximum(cn, 1.0)
    y = lax.dot_general(m, w_ref[...], (((1,), (1,)), ((), ())),
                        preferred_element_type=jnp.float32)
    y = y + b_ref[...]
    z = y - jnp.max(y, axis=1, keepdims=True)
    out_ref[...] = z - jnp.log(jnp.sum(jnp.exp(z), axis=1, keepdims=True))


_TC1_BLK = 1264

_tc1 = pl.pallas_call(
    _tc1_body,
    grid=(_NPAD1 // _TC1_BLK,),
    in_specs=[
        pl.BlockSpec((_NC, _TC1_BLK, _DH), lambda i: (0, i, 0)),
        pl.BlockSpec((_NC, _TC1_BLK, _CW), lambda i: (0, i, 0)),
        pl.BlockSpec((_D, _D), lambda i: (0, 0)),
        pl.BlockSpec((1, _D), lambda i: (0, 0)),
    ],
    out_specs=pl.BlockSpec((_TC1_BLK, _D), lambda i: (i, 0)),
    out_shape=jax.ShapeDtypeStruct((_NPAD1, _D), jnp.float32),
)

_tc2 = pl.pallas_call(
    _tc2_body,
    in_specs=[
        pl.BlockSpec((_NC, _NPAD2, _DH), lambda: (0, 0, 0)),
        pl.BlockSpec((_NC, _NPAD2, _CW), lambda: (0, 0, 0)),
        pl.BlockSpec((64, _D), lambda: (0, 0)),
        pl.BlockSpec((1, 64), lambda: (0, 0)),
    ],
    out_specs=pl.BlockSpec((_NPAD2, 64), lambda: (0, 0)),
    out_shape=jax.ShapeDtypeStruct((_NPAD2, 64), jnp.float32),
)


def _pad_edges(edge_index, e_pad, trash_row):
    npad_e = e_pad - edge_index.shape[1]
    src = jnp.concatenate(
        [edge_index[0], jnp.zeros((npad_e,), jnp.int32)]).reshape(-1, _B)
    dst = jnp.concatenate(
        [edge_index[1],
         jnp.full((npad_e,), trash_row, jnp.int32)]).reshape(-1, _B)
    return src, dst


def kernel(x, edge_index1, edge_index2, size1, size2, W1, b1, W2, b2):
    src1, dst1 = _pad_edges(edge_index1, _E1_PAD, _NPAD1 - 1)
    acc1 = _sc_agg1(x.reshape(-1, _DH), src1, dst1)
    cnt1 = _sc_cnt1(dst1)
    h = _tc1(acc1.reshape(_NC, _NPAD1, _DH), cnt1.reshape(_NC, _NPAD1, _CW),
             W1, b1.reshape(1, _D))
    src2, dst2 = _pad_edges(edge_index2, _E2_PAD, _NPAD2 - 1)
    acc2 = _sc_agg2(h.reshape(-1, _DH), src2, dst2)
    cnt2 = _sc_cnt2(dst2)
    out = _tc2(acc2.reshape(_NC, _NPAD2, _DH), cnt2.reshape(_NC, _NPAD2, _CW),
               W2, b2.reshape(1, 64))
    return out[:_N2]


# speedup vs baseline: 2.0275x; 2.0275x over previous
import functools

import jax
import jax.numpy as jnp
from jax import lax
from jax.experimental import pallas as pl
from jax.experimental.pallas import tpu as pltpu
from jax.experimental.pallas import tpu_sc as plsc

_NC = 2
_NS = 16
_NW = _NC * _NS
_D = 128
_DH = 64
_CW = 32

_N1, _N2 = 10000, 2000
_NPAD1, _NPAD2 = 10112, 2048


def _make_sc_agg(E, NPAD, B, K):
    e_per_t = E // _NS
    nb = e_per_t // B
    ns = nb // K
    assert nb * B == e_per_t and B % 16 == 0 and B <= 128 and ns * K == nb
    rpt = NPAD // _NS
    ZR = rpt // 2
    assert rpt % ZR == 0 and NPAD % 16 == 0 and ZR % 4 == 0
    mesh = plsc.VectorSubcoreMesh(core_axis_name="c", subcore_axis_name="s")

    @functools.partial(
        pl.kernel,
        out_type=jax.ShapeDtypeStruct((_NC * NPAD, _DH), jnp.float32),
        mesh=mesh,
        compiler_params=pltpu.CompilerParams(use_tc_tiling_on_sc=False),
        scratch_types=[
            pltpu.VMEM((K, B), jnp.int32),
            pltpu.VMEM((K, B), jnp.int32),
            pltpu.VMEM((K, B), jnp.int32),
            pltpu.VMEM((K, B, _DH), jnp.float32),
            pltpu.VMEM((ZR, _DH), jnp.float32),
            pltpu.VMEM_SHARED((NPAD, _DH), jnp.float32),
            pltpu.SemaphoreType.DMA,
            pltpu.SemaphoreType.DMA,
        ],
    )
    def agg(tab, src, dst, out_acc, idx_s, idx_d, idx_g, rows, zrow,
            acc_sh, sem_g, sem_s):
        c = lax.axis_index("c")
        s = lax.axis_index("s")

        z16 = jnp.zeros((16,), jnp.float32)

        def zr_body(i, carry):
            zrow[i // 4, pl.ds((i % 4) * 16, 16)] = z16
            return carry

        lax.fori_loop(0, ZR * 4, zr_body, 0)

        for k in range(rpt // ZR):
            pltpu.sync_copy(zrow, acc_sh.at[pl.ds(s * rpt + k * ZR, ZR)])
        plsc.subcore_barrier()

        row0 = s * nb

        def body(i, carry):
            r = row0 + i * K
            pltpu.sync_copy(src.at[pl.ds(r, K)], idx_s)
            pltpu.sync_copy(dst.at[pl.ds(r, K)], idx_d)
            for k in range(K):
                for j in range(B // 16):
                    v = idx_s[k, pl.ds(j * 16, 16)]
                    idx_g[k, pl.ds(j * 16, 16)] = v * 2 + c
            gets = [pltpu.async_copy(tab.at[idx_g.at[k]], rows.at[k], sem_g)
                    for k in range(K)]
            puts = []
            for k in range(K):
                gets[k].wait()
                puts.append(pltpu.async_copy(
                    rows.at[k], acc_sh.at[idx_d.at[k]], sem_s, add=True))
            for p in puts:
                p.wait()
            return carry

        lax.fori_loop(0, ns, body, 0)
        plsc.subcore_barrier()

        pltpu.sync_copy(acc_sh.at[pl.ds(s * rpt, rpt)],
                        out_acc.at[pl.ds(c * NPAD + s * rpt, rpt)])

    return agg


def _make_sc_cnt(E, NPAD, B, K):
    e_per_w = E // _NW
    nb = e_per_w // B
    ns = nb // K
    assert nb * B == e_per_w and B % 16 == 0 and B <= 128 and ns * K == nb
    rpt = NPAD // _NS
    mesh = plsc.VectorSubcoreMesh(core_axis_name="c", subcore_axis_name="s")

    @functools.partial(
        pl.kernel,
        out_type=jax.ShapeDtypeStruct((_NC * NPAD, _CW), jnp.float32),
        mesh=mesh,
        compiler_params=pltpu.CompilerParams(use_tc_tiling_on_sc=False),
        scratch_types=[
            pltpu.VMEM((K, B), jnp.int32),
            pltpu.VMEM((B, _CW), jnp.float32),
            pltpu.VMEM((rpt, _CW), jnp.float32),
            pltpu.VMEM_SHARED((NPAD, _CW), jnp.float32),
            pltpu.SemaphoreType.DMA,
        ],
    )
    def cnt(dst, out_cnt, idx_d, ones2, zc, cnt_sh, sem_s):
        c = lax.axis_index("c")
        s = lax.axis_index("s")
        wid = s * _NC + c

        z16 = jnp.zeros((16,), jnp.float32)
        one16 = jnp.ones((16,), jnp.float32)

        def zb(i, carry):
            zc[i, pl.ds(0, 16)] = z16
            zc[i, pl.ds(16, 16)] = z16
            return carry

        lax.fori_loop(0, rpt, zb, 0)

        def ob(i, carry):
            ones2[i, pl.ds(0, 16)] = one16
            ones2[i, pl.ds(16, 16)] = one16
            return carry

        lax.fori_loop(0, B, ob, 0)

        pltpu.sync_copy(zc, cnt_sh.at[pl.ds(s * rpt, rpt)])
        plsc.subcore_barrier()

        row0 = wid * nb

        def body(i, carry):
            pltpu.sync_copy(dst.at[pl.ds(row0 + i * K, K)], idx_d)
            puts = [pltpu.async_copy(ones2, cnt_sh.at[idx_d.at[k]], sem_s,
                                     add=True)
                    for k in range(K)]
            for p in puts:
                p.wait()
            return carry

        lax.fori_loop(0, ns, body, 0)
        plsc.subcore_barrier()

        pltpu.sync_copy(cnt_sh.at[pl.ds(s * rpt, rpt)],
                        out_cnt.at[pl.ds(c * NPAD + s * rpt, rpt)])

    return cnt


_E1_PAD = 320000
_E2_PAD = 161280
_B = 80
_sc_agg1 = _make_sc_agg(_E1_PAD, _NPAD1, _B, 10)
_sc_agg2 = _make_sc_agg(_E2_PAD, _NPAD2, _B, 9)
_sc_cnt1 = _make_sc_cnt(_E1_PAD, _NPAD1, _B, 5)
_sc_cnt2 = _make_sc_cnt(_E2_PAD, _NPAD2, _B, 9)


def _tc1_body(acc_ref, cnt_ref, w_ref, b_ref, out_ref):
    a = jnp.concatenate([acc_ref[0], acc_ref[1]], axis=1)
    cn = cnt_ref[0, :, 0:1] + cnt_ref[1, :, 0:1]
    m = a / jnp.maximum(cn, 1.0)
    y = lax.dot_general(m, w_ref[...], (((1,), (1,)), ((), ())),
                        preferred_element_type=jnp.float32)
    out_ref[...] = jnp.maximum(y + b_ref[...], 0.0)


def _tc2_body(acc_ref, cnt_ref, w_ref, b_ref, out_ref):
    a = jnp.concatenate([acc_ref[0], acc_ref[1]], axis=1)
    cn = cnt_ref[0, :, 0:1] + cnt_ref[1, :, 0:1]
    m = a / jnp.maximum(cn, 1.0)
    y = lax.dot_general(m, w_ref[...], (((1,), (1,)), ((), ())),
                        preferred_element_type=jnp.float32)
    y = y + b_ref[...]
    z = y - jnp.max(y, axis=1, keepdims=True)
    out_ref[...] = z - jnp.log(jnp.sum(jnp.exp(z), axis=1, keepdims=True))


_TC1_BLK = 1264

_tc1 = pl.pallas_call(
    _tc1_body,
    grid=(_NPAD1 // _TC1_BLK,),
    in_specs=[
        pl.BlockSpec((_NC, _TC1_BLK, _DH), lambda i: (0, i, 0)),
        pl.BlockSpec((_NC, _TC1_BLK, _CW), lambda i: (0, i, 0)),
        pl.BlockSpec((_D, _D), lambda i: (0, 0)),
        pl.BlockSpec((1, _D), lambda i: (0, 0)),
    ],
    out_specs=pl.BlockSpec((_TC1_BLK, _D), lambda i: (i, 0)),
    out_shape=jax.ShapeDtypeStruct((_NPAD1, _D), jnp.float32),
)

_tc2 = pl.pallas_call(
    _tc2_body,
    in_specs=[
        pl.BlockSpec((_NC, _NPAD2, _DH), lambda: (0, 0, 0)),
        pl.BlockSpec((_NC, _NPAD2, _CW), lambda: (0, 0, 0)),
        pl.BlockSpec((64, _D), lambda: (0, 0)),
        pl.BlockSpec((1, 64), lambda: (0, 0)),
    ],
    out_specs=pl.BlockSpec((_NPAD2, 64), lambda: (0, 0)),
    out_shape=jax.ShapeDtypeStruct((_NPAD2, 64), jnp.float32),
)


def _pad_edges(edge_index, e_pad, trash_row):
    npad_e = e_pad - edge_index.shape[1]
    src = jnp.concatenate(
        [edge_index[0], jnp.zeros((npad_e,), jnp.int32)]).reshape(-1, _B)
    dst = jnp.concatenate(
        [edge_index[1],
         jnp.full((npad_e,), trash_row, jnp.int32)]).reshape(-1, _B)
    return src, dst


def kernel(x, edge_index1, edge_index2, size1, size2, W1, b1, W2, b2):
    src1, dst1 = _pad_edges(edge_index1, _E1_PAD, _NPAD1 - 1)
    acc1 = _sc_agg1(x.reshape(-1, _DH), src1, dst1)
    cnt1 = _sc_cnt1(dst1)
    h = _tc1(acc1.reshape(_NC, _NPAD1, _DH), cnt1.reshape(_NC, _NPAD1, _CW),
             W1, b1.reshape(1, _D))
    src2, dst2 = _pad_edges(edge_index2, _E2_PAD, _NPAD2 - 1)
    acc2 = _sc_agg2(h.reshape(-1, _DH), src2, dst2)
    cnt2 = _sc_cnt2(dst2)
    out = _tc2(acc2.reshape(_NC, _NPAD2, _DH), cnt2.reshape(_NC, _NPAD2, _CW),
               W2, b2.reshape(1, 64))
    return out[:_N2]


# speedup vs baseline: 2.2675x; 1.1184x over previous
import functools

import jax
import jax.numpy as jnp
from jax import lax
from jax.experimental import pallas as pl
from jax.experimental.pallas import tpu as pltpu
from jax.experimental.pallas import tpu_sc as plsc

_NC = 2
_NS = 16
_NW = _NC * _NS
_D = 128
_DH = 64
_CW = 32

_N1, _N2 = 10000, 2000
_NPAD1, _NPAD2 = 10112, 2048


def _make_sc_agg(E, NPAD, B, K):
    e_per_t = E // _NS
    nb = e_per_t // B
    ns = nb // K
    assert nb * B == e_per_t and B % 16 == 0 and B <= 128 and ns * K == nb
    rpt = NPAD // _NS
    ZR = rpt // 2
    assert rpt % ZR == 0 and NPAD % 16 == 0 and ZR % 4 == 0
    mesh = plsc.VectorSubcoreMesh(core_axis_name="c", subcore_axis_name="s")

    @functools.partial(
        pl.kernel,
        out_type=jax.ShapeDtypeStruct((_NC * NPAD, _DH), jnp.float32),
        mesh=mesh,
        compiler_params=pltpu.CompilerParams(use_tc_tiling_on_sc=False),
        scratch_types=[
            pltpu.VMEM((2 * K, B), jnp.int32),
            pltpu.VMEM((2 * K, B), jnp.int32),
            pltpu.VMEM((2 * K, B), jnp.int32),
            pltpu.VMEM((2 * K, B, _DH), jnp.float32),
            pltpu.VMEM((ZR, _DH), jnp.float32),
            pltpu.VMEM_SHARED((NPAD, _DH), jnp.float32),
            pltpu.SemaphoreType.DMA,
            pltpu.SemaphoreType.DMA,
            pltpu.SemaphoreType.DMA,
        ],
    )
    def agg(tab, src, dst, out_acc, idx_s, idx_d, idx_g, rows, zrow,
            acc_sh, sem_g, sem_s, sem_i):
        c = lax.axis_index("c")
        s = lax.axis_index("s")

        z16 = jnp.zeros((16,), jnp.float32)

        def zr_body(i, carry):
            zrow[i // 4, pl.ds((i % 4) * 16, 16)] = z16
            return carry

        lax.fori_loop(0, ZR * 4, zr_body, 0)

        for k in range(rpt // ZR):
            pltpu.sync_copy(zrow, acc_sh.at[pl.ds(s * rpt + k * ZR, ZR)])
        plsc.subcore_barrier()

        row0 = s * nb

        def compute_idx_g(base_slot):
            for k in range(K):
                t = base_slot + k
                for j in range(B // 16):
                    v = idx_s[t, pl.ds(j * 16, 16)]
                    idx_g[t, pl.ds(j * 16, 16)] = v * 2 + c

        def fire_gathers(base_slot):
            for k in range(K):
                t = base_slot + k
                pltpu.async_copy(tab.at[idx_g.at[t]], rows.at[t], sem_g)

        def drain_scatters(base_slot):
            for k in range(K):
                t = base_slot + k
                pltpu.make_async_copy(
                    rows.at[t], acc_sh.at[idx_d.at[t]], sem_s).wait()

        pltpu.sync_copy(src.at[pl.ds(row0, K)], idx_s.at[pl.ds(0, K)])
        pltpu.sync_copy(dst.at[pl.ds(row0, K)], idx_d.at[pl.ds(0, K)])
        compute_idx_g(0)
        fire_gathers(0)

        def body(i, carry):
            p = lax.rem(i, 2) * K
            pn = (1 - lax.rem(i, 2)) * K
            r_next = row0 + (i + 1) * K

            @pl.when(i >= 1)
            def _():
                drain_scatters(pn)

            @pl.when(i + 1 < ns)
            def _():
                pltpu.async_copy(src.at[pl.ds(r_next, K)],
                                 idx_s.at[pl.ds(pn, K)], sem_i)
                pltpu.async_copy(dst.at[pl.ds(r_next, K)],
                                 idx_d.at[pl.ds(pn, K)], sem_i)

            for k in range(K):
                t = p + k
                pltpu.make_async_copy(
                    tab.at[idx_g.at[t]], rows.at[t], sem_g).wait()
                pltpu.async_copy(
                    rows.at[t], acc_sh.at[idx_d.at[t]], sem_s, add=True)

            @pl.when(i + 1 < ns)
            def _():
                pltpu.make_async_copy(src.at[pl.ds(r_next, K)],
                                      idx_s.at[pl.ds(pn, K)], sem_i).wait()
                pltpu.make_async_copy(dst.at[pl.ds(r_next, K)],
                                      idx_d.at[pl.ds(pn, K)], sem_i).wait()
                compute_idx_g(pn)
                fire_gathers(pn)

            return carry

        lax.fori_loop(0, ns, body, 0)
        drain_scatters(((ns - 1) % 2) * K)
        plsc.subcore_barrier()

        pltpu.sync_copy(acc_sh.at[pl.ds(s * rpt, rpt)],
                        out_acc.at[pl.ds(c * NPAD + s * rpt, rpt)])

    return agg


def _make_sc_cnt(E, NPAD, B, K):
    e_per_w = E // _NW
    nb = e_per_w // B
    ns = nb // K
    assert nb * B == e_per_w and B % 16 == 0 and B <= 128 and ns * K == nb
    rpt = NPAD // _NS
    mesh = plsc.VectorSubcoreMesh(core_axis_name="c", subcore_axis_name="s")

    @functools.partial(
        pl.kernel,
        out_type=jax.ShapeDtypeStruct((_NC * NPAD, _CW), jnp.float32),
        mesh=mesh,
        compiler_params=pltpu.CompilerParams(use_tc_tiling_on_sc=False),
        scratch_types=[
            pltpu.VMEM((K, B), jnp.int32),
            pltpu.VMEM((B, _CW), jnp.float32),
            pltpu.VMEM((rpt, _CW), jnp.float32),
            pltpu.VMEM_SHARED((NPAD, _CW), jnp.float32),
            pltpu.SemaphoreType.DMA,
        ],
    )
    def cnt(dst, out_cnt, idx_d, ones2, zc, cnt_sh, sem_s):
        c = lax.axis_index("c")
        s = lax.axis_index("s")
        wid = s * _NC + c

        z16 = jnp.zeros((16,), jnp.float32)
        one16 = jnp.ones((16,), jnp.float32)

        def zb(i, carry):
            zc[i, pl.ds(0, 16)] = z16
            zc[i, pl.ds(16, 16)] = z16
            return carry

        lax.fori_loop(0, rpt, zb, 0)

        def ob(i, carry):
            ones2[i, pl.ds(0, 16)] = one16
            ones2[i, pl.ds(16, 16)] = one16
            return carry

        lax.fori_loop(0, B, ob, 0)

        pltpu.sync_copy(zc, cnt_sh.at[pl.ds(s * rpt, rpt)])
        plsc.subcore_barrier()

        row0 = wid * nb

        def body(i, carry):
            pltpu.sync_copy(dst.at[pl.ds(row0 + i * K, K)], idx_d)
            puts = [pltpu.async_copy(ones2, cnt_sh.at[idx_d.at[k]], sem_s,
                                     add=True)
                    for k in range(K)]
            for p in puts:
                p.wait()
            return carry

        lax.fori_loop(0, ns, body, 0)
        plsc.subcore_barrier()

        pltpu.sync_copy(cnt_sh.at[pl.ds(s * rpt, rpt)],
                        out_cnt.at[pl.ds(c * NPAD + s * rpt, rpt)])

    return cnt


_E1_PAD = 320000
_E2_PAD = 161280
_B = 80
_sc_agg1 = _make_sc_agg(_E1_PAD, _NPAD1, _B, 5)
_sc_agg2 = _make_sc_agg(_E2_PAD, _NPAD2, _B, 7)
_sc_cnt1 = _make_sc_cnt(_E1_PAD, _NPAD1, _B, 5)
_sc_cnt2 = _make_sc_cnt(_E2_PAD, _NPAD2, _B, 9)


def _tc1_body(acc_ref, cnt_ref, w_ref, b_ref, out_ref):
    a = jnp.concatenate([acc_ref[0], acc_ref[1]], axis=1)
    cn = cnt_ref[0, :, 0:1] + cnt_ref[1, :, 0:1]
    m = a / jnp.maximum(cn, 1.0)
    y = lax.dot_general(m, w_ref[...], (((1,), (1,)), ((), ())),
                        preferred_element_type=jnp.float32)
    out_ref[...] = jnp.maximum(y + b_ref[...], 0.0)


def _tc2_body(acc_ref, cnt_ref, w_ref, b_ref, out_ref):
    a = jnp.concatenate([acc_ref[0], acc_ref[1]], axis=1)
    cn = cnt_ref[0, :, 0:1] + cnt_ref[1, :, 0:1]
    m = a / jnp.maximum(cn, 1.0)
    y = lax.dot_general(m, w_ref[...], (((1,), (1,)), ((), ())),
                        preferred_element_type=jnp.float32)
    y = y + b_ref[...]
    z = y - jnp.max(y, axis=1, keepdims=True)
    out_ref[...] = z - jnp.log(jnp.sum(jnp.exp(z), axis=1, keepdims=True))


_TC1_BLK = 1264

_tc1 = pl.pallas_call(
    _tc1_body,
    grid=(_NPAD1 // _TC1_BLK,),
    in_specs=[
        pl.BlockSpec((_NC, _TC1_BLK, _DH), lambda i: (0, i, 0)),
        pl.BlockSpec((_NC, _TC1_BLK, _CW), lambda i: (0, i, 0)),
        pl.BlockSpec((_D, _D), lambda i: (0, 0)),
        pl.BlockSpec((1, _D), lambda i: (0, 0)),
    ],
    out_specs=pl.BlockSpec((_TC1_BLK, _D), lambda i: (i, 0)),
    out_shape=jax.ShapeDtypeStruct((_NPAD1, _D), jnp.float32),
)

_tc2 = pl.pallas_call(
    _tc2_body,
    in_specs=[
        pl.BlockSpec((_NC, _NPAD2, _DH), lambda: (0, 0, 0)),
        pl.BlockSpec((_NC, _NPAD2, _CW), lambda: (0, 0, 0)),
        pl.BlockSpec((64, _D), lambda: (0, 0)),
        pl.BlockSpec((1, 64), lambda: (0, 0)),
    ],
    out_specs=pl.BlockSpec((_NPAD2, 64), lambda: (0, 0)),
    out_shape=jax.ShapeDtypeStruct((_NPAD2, 64), jnp.float32),
)


def _pad_edges(edge_index, e_pad, trash_row):
    npad_e = e_pad - edge_index.shape[1]
    src = jnp.concatenate(
        [edge_index[0], jnp.zeros((npad_e,), jnp.int32)]).reshape(-1, _B)
    dst = jnp.concatenate(
        [edge_index[1],
         jnp.full((npad_e,), trash_row, jnp.int32)]).reshape(-1, _B)
    return src, dst


def kernel(x, edge_index1, edge_index2, size1, size2, W1, b1, W2, b2):
    src1, dst1 = _pad_edges(edge_index1, _E1_PAD, _NPAD1 - 1)
    acc1 = _sc_agg1(x.reshape(-1, _DH), src1, dst1)
    cnt1 = _sc_cnt1(dst1)
    h = _tc1(acc1.reshape(_NC, _NPAD1, _DH), cnt1.reshape(_NC, _NPAD1, _CW),
             W1, b1.reshape(1, _D))
    src2, dst2 = _pad_edges(edge_index2, _E2_PAD, _NPAD2 - 1)
    acc2 = _sc_agg2(h.reshape(-1, _DH), src2, dst2)
    cnt2 = _sc_cnt2(dst2)
    out = _tc2(acc2.reshape(_NC, _NPAD2, _DH), cnt2.reshape(_NC, _NPAD2, _CW),
               W2, b2.reshape(1, 64))
    return out[:_N2]


# speedup vs baseline: 2.3553x; 1.0387x over previous
import functools

import jax
import jax.numpy as jnp
from jax import lax
from jax.experimental import pallas as pl
from jax.experimental.pallas import tpu as pltpu
from jax.experimental.pallas import tpu_sc as plsc

_NC = 2
_NS = 16
_NW = _NC * _NS
_D = 128
_DH = 64
_CW = 32

_N1, _N2 = 10000, 2000
_NPAD1, _NPAD2 = 10112, 2048


def _make_sc_agg(E, NPAD, B, K):
    e_per_t = E // _NS
    nb = e_per_t // B
    ns = nb // K
    assert nb * B == e_per_t and B % 16 == 0 and B <= 128 and ns * K == nb
    rpt = NPAD // _NS
    ZR = rpt // 2
    assert rpt % ZR == 0 and NPAD % 16 == 0 and ZR % 4 == 0
    mesh = plsc.VectorSubcoreMesh(core_axis_name="c", subcore_axis_name="s")

    @functools.partial(
        pl.kernel,
        out_type=jax.ShapeDtypeStruct((_NC * NPAD, _DH), jnp.float32),
        mesh=mesh,
        compiler_params=pltpu.CompilerParams(use_tc_tiling_on_sc=False),
        scratch_types=[
            pltpu.VMEM((2 * K, B), jnp.int32),
            pltpu.VMEM((2 * K, B), jnp.int32),
            pltpu.VMEM((2 * K, B), jnp.int32),
            pltpu.VMEM((2 * K, B, _DH), jnp.float32),
            pltpu.VMEM((ZR, _DH), jnp.float32),
            pltpu.VMEM_SHARED((NPAD, _DH), jnp.float32),
            pltpu.SemaphoreType.DMA,
            pltpu.SemaphoreType.DMA,
            pltpu.SemaphoreType.DMA,
        ],
    )
    def agg(tab, src, dst, out_acc, idx_s, idx_d, idx_g, rows, zrow,
            acc_sh, sem_g, sem_s, sem_i):
        c = lax.axis_index("c")
        s = lax.axis_index("s")

        z16 = jnp.zeros((16,), jnp.float32)

        def zr_body(i, carry):
            zrow[i // 4, pl.ds((i % 4) * 16, 16)] = z16
            return carry

        lax.fori_loop(0, ZR * 4, zr_body, 0)

        for k in range(rpt // ZR):
            pltpu.sync_copy(zrow, acc_sh.at[pl.ds(s * rpt + k * ZR, ZR)])
        plsc.subcore_barrier()

        row0 = s * nb

        def compute_idx_g(base_slot):
            for k in range(K):
                t = base_slot + k
                for j in range(B // 16):
                    v = idx_s[t, pl.ds(j * 16, 16)]
                    idx_g[t, pl.ds(j * 16, 16)] = v * 2 + c

        def fire_gathers(base_slot):
            for k in range(K):
                t = base_slot + k
                pltpu.async_copy(tab.at[idx_g.at[t]], rows.at[t], sem_g)

        def drain_scatters(base_slot):
            for k in range(K):
                t = base_slot + k
                pltpu.make_async_copy(
                    rows.at[t], acc_sh.at[idx_d.at[t]], sem_s).wait()

        pltpu.sync_copy(src.at[pl.ds(row0, K)], idx_s.at[pl.ds(0, K)])
        pltpu.sync_copy(dst.at[pl.ds(row0, K)], idx_d.at[pl.ds(0, K)])
        compute_idx_g(0)
        fire_gathers(0)

        def body(i, carry):
            p = lax.rem(i, 2) * K
            pn = (1 - lax.rem(i, 2)) * K
            r_next = row0 + (i + 1) * K

            @pl.when(i >= 1)
            def _():
                drain_scatters(pn)

            @pl.when(i + 1 < ns)
            def _():
                pltpu.async_copy(src.at[pl.ds(r_next, K)],
                                 idx_s.at[pl.ds(pn, K)], sem_i)
                pltpu.async_copy(dst.at[pl.ds(r_next, K)],
                                 idx_d.at[pl.ds(pn, K)], sem_i)

            for k in range(K):
                t = p + k
                pltpu.make_async_copy(
                    tab.at[idx_g.at[t]], rows.at[t], sem_g).wait()
                pltpu.async_copy(
                    rows.at[t], acc_sh.at[idx_d.at[t]], sem_s, add=True)

            @pl.when(i + 1 < ns)
            def _():
                pltpu.make_async_copy(src.at[pl.ds(r_next, K)],
                                      idx_s.at[pl.ds(pn, K)], sem_i).wait()
                pltpu.make_async_copy(dst.at[pl.ds(r_next, K)],
                                      idx_d.at[pl.ds(pn, K)], sem_i).wait()
                compute_idx_g(pn)
                fire_gathers(pn)

            return carry

        lax.fori_loop(0, ns, body, 0)
        drain_scatters(((ns - 1) % 2) * K)
        plsc.subcore_barrier()

        pltpu.sync_copy(acc_sh.at[pl.ds(s * rpt, rpt)],
                        out_acc.at[pl.ds(c * NPAD + s * rpt, rpt)])

    return agg


def _make_sc_cnt(E, NPAD, B, K):
    e_per_w = E // _NW
    nb = e_per_w // B
    ns = nb // K
    assert nb * B == e_per_w and B % 16 == 0 and B <= 128 and ns * K == nb
    rpt = NPAD // _NS
    mesh = plsc.VectorSubcoreMesh(core_axis_name="c", subcore_axis_name="s")

    @functools.partial(
        pl.kernel,
        out_type=jax.ShapeDtypeStruct((_NC * NPAD, _CW), jnp.float32),
        mesh=mesh,
        compiler_params=pltpu.CompilerParams(use_tc_tiling_on_sc=False),
        scratch_types=[
            pltpu.VMEM((2 * K, B), jnp.int32),
            pltpu.VMEM((B, _CW), jnp.float32),
            pltpu.VMEM((rpt, _CW), jnp.float32),
            pltpu.VMEM_SHARED((NPAD, _CW), jnp.float32),
            pltpu.SemaphoreType.DMA,
            pltpu.SemaphoreType.DMA,
        ],
    )
    def cnt(dst, out_cnt, idx_d, ones2, zc, cnt_sh, sem_s, sem_i):
        c = lax.axis_index("c")
        s = lax.axis_index("s")
        wid = s * _NC + c

        z16 = jnp.zeros((16,), jnp.float32)
        one16 = jnp.ones((16,), jnp.float32)

        def zb(i, carry):
            zc[i, pl.ds(0, 16)] = z16
            zc[i, pl.ds(16, 16)] = z16
            return carry

        lax.fori_loop(0, rpt, zb, 0)

        def ob(i, carry):
            ones2[i, pl.ds(0, 16)] = one16
            ones2[i, pl.ds(16, 16)] = one16
            return carry

        lax.fori_loop(0, B, ob, 0)

        pltpu.sync_copy(zc, cnt_sh.at[pl.ds(s * rpt, rpt)])
        plsc.subcore_barrier()

        row0 = wid * nb

        def drain_scatters(base_slot):
            for k in range(K):
                pltpu.make_async_copy(
                    ones2, cnt_sh.at[idx_d.at[base_slot + k]], sem_s).wait()

        pltpu.sync_copy(dst.at[pl.ds(row0, K)], idx_d.at[pl.ds(0, K)])

        def body(i, carry):
            p = lax.rem(i, 2) * K
            pn = (1 - lax.rem(i, 2)) * K
            r_next = row0 + (i + 1) * K

            @pl.when(i >= 1)
            def _():
                drain_scatters(pn)

            @pl.when(i + 1 < ns)
            def _():
                pltpu.async_copy(dst.at[pl.ds(r_next, K)],
                                 idx_d.at[pl.ds(pn, K)], sem_i)

            for k in range(K):
                pltpu.async_copy(ones2, cnt_sh.at[idx_d.at[p + k]], sem_s,
                                 add=True)

            @pl.when(i + 1 < ns)
            def _():
                pltpu.make_async_copy(dst.at[pl.ds(r_next, K)],
                                      idx_d.at[pl.ds(pn, K)], sem_i).wait()

            return carry

        lax.fori_loop(0, ns, body, 0)
        drain_scatters(((ns - 1) % 2) * K)
        plsc.subcore_barrier()

        pltpu.sync_copy(cnt_sh.at[pl.ds(s * rpt, rpt)],
                        out_cnt.at[pl.ds(c * NPAD + s * rpt, rpt)])

    return cnt


_E1_PAD = 320000
_E2_PAD = 161280
_B = 80
_sc_agg1 = _make_sc_agg(_E1_PAD, _NPAD1, _B, 5)
_sc_agg2 = _make_sc_agg(_E2_PAD, _NPAD2, _B, 7)
_sc_cnt1 = _make_sc_cnt(_E1_PAD, _NPAD1, _B, 5)
_sc_cnt2 = _make_sc_cnt(_E2_PAD, _NPAD2, _B, 9)


def _tc1_body(acc_ref, cnt_ref, w_ref, b_ref, out_ref):
    a = jnp.concatenate([acc_ref[0], acc_ref[1]], axis=1)
    cn = cnt_ref[0, :, 0:1] + cnt_ref[1, :, 0:1]
    m = a / jnp.maximum(cn, 1.0)
    y = lax.dot_general(m, w_ref[...], (((1,), (1,)), ((), ())),
                        preferred_element_type=jnp.float32)
    out_ref[...] = jnp.maximum(y + b_ref[...], 0.0)


def _tc2_body(acc_ref, cnt_ref, w_ref, b_ref, out_ref):
    a = jnp.concatenate([acc_ref[0], acc_ref[1]], axis=1)
    cn = cnt_ref[0, :, 0:1] + cnt_ref[1, :, 0:1]
    m = a / jnp.maximum(cn, 1.0)
    y = lax.dot_general(m, w_ref[...], (((1,), (1,)), ((), ())),
                        preferred_element_type=jnp.float32)
    y = y + b_ref[...]
    z = y - jnp.max(y, axis=1, keepdims=True)
    out_ref[...] = z - jnp.log(jnp.sum(jnp.exp(z), axis=1, keepdims=True))


_TC1_BLK = 1264

_tc1 = pl.pallas_call(
    _tc1_body,
    grid=(_NPAD1 // _TC1_BLK,),
    in_specs=[
        pl.BlockSpec((_NC, _TC1_BLK, _DH), lambda i: (0, i, 0)),
        pl.BlockSpec((_NC, _TC1_BLK, _CW), lambda i: (0, i, 0)),
        pl.BlockSpec((_D, _D), lambda i: (0, 0)),
        pl.BlockSpec((1, _D), lambda i: (0, 0)),
    ],
    out_specs=pl.BlockSpec((_TC1_BLK, _D), lambda i: (i, 0)),
    out_shape=jax.ShapeDtypeStruct((_NPAD1, _D), jnp.float32),
)

_tc2 = pl.pallas_call(
    _tc2_body,
    in_specs=[
        pl.BlockSpec((_NC, _NPAD2, _DH), lambda: (0, 0, 0)),
        pl.BlockSpec((_NC, _NPAD2, _CW), lambda: (0, 0, 0)),
        pl.BlockSpec((64, _D), lambda: (0, 0)),
        pl.BlockSpec((1, 64), lambda: (0, 0)),
    ],
    out_specs=pl.BlockSpec((_NPAD2, 64), lambda: (0, 0)),
    out_shape=jax.ShapeDtypeStruct((_NPAD2, 64), jnp.float32),
)


def _pad_edges(edge_index, e_pad, n_real, npad):
    npad_e = e_pad - edge_index.shape[1]
    src = jnp.concatenate(
        [edge_index[0], jnp.zeros((npad_e,), jnp.int32)]).reshape(-1, _B)
    trash = n_real + jnp.arange(npad_e, dtype=jnp.int32) % (npad - n_real)
    dst = jnp.concatenate([edge_index[1], trash]).reshape(-1, _B)
    return src, dst


def kernel(x, edge_index1, edge_index2, size1, size2, W1, b1, W2, b2):
    src1, dst1 = _pad_edges(edge_index1, _E1_PAD, _N1, _NPAD1)
    acc1 = _sc_agg1(x.reshape(-1, _DH), src1, dst1)
    cnt1 = _sc_cnt1(dst1)
    h = _tc1(acc1.reshape(_NC, _NPAD1, _DH), cnt1.reshape(_NC, _NPAD1, _CW),
             W1, b1.reshape(1, _D))
    src2, dst2 = _pad_edges(edge_index2, _E2_PAD, _N2, _NPAD2)
    acc2 = _sc_agg2(h.reshape(-1, _DH), src2, dst2)
    cnt2 = _sc_cnt2(dst2)
    out = _tc2(acc2.reshape(_NC, _NPAD2, _DH), cnt2.reshape(_NC, _NPAD2, _CW),
               W2, b2.reshape(1, 64))
    return out[:_N2]
